# baseline (speedup 1.0000x reference)
import functools

import jax
import jax.numpy as jnp
from jax import lax
from jax.experimental import pallas as pl
from jax.experimental.pallas import tpu as pltpu
from jax.experimental.pallas import tpu_sc as plsc

_NUM_CORES = 2
_NUM_SUBCORES = 16
_LANES = 16
_NW = _NUM_CORES * _NUM_SUBCORES

_CHUNK = 8192


def _fold(x):
    u = jnp.abs(lax.rem(x, jnp.float32(2.0)))
    return jnp.where(u <= jnp.float32(1.0), u, jnp.float32(2.0) - u)


def _sc_body(n_rays, table_hbm, r_hbm, g_hbm, b_hbm, idx_hbm,
             o0_hbm, o1_hbm, o2_hbm,
             table_v, idx_v, r_v, g_v, b_v, o0_v, o1_v, o2_v,
             in_sems, out_sems):
    rays_per_worker = n_rays // _NW
    chunks = rays_per_worker // _CHUNK
    groups = _CHUNK // _LANES

    wid = lax.axis_index("s") * _NUM_CORES + lax.axis_index("c")
    worker_base = wid * rays_per_worker

    def run_groups(buf):
        @plsc.parallel_loop(0, groups, step=1, unroll=4)
        def _(g):
            s = pl.ds(g * _LANES, _LANES)
            vidx = idx_v[buf][s]
            base = vidx * 6
            p = []
            for k in range(6):
                w = plsc.load_gather(table_v, [base + k])
                pair = plsc.unpack(plsc.bitcast(w, jnp.bfloat16),
                                   format=plsc.PackFormat.INTERLEAVED)
                p.extend(pair)
            r = r_v[buf][s]
            gg = g_v[buf][s]
            b = b_v[buf][s]
            o0_v[buf][s] = _fold(p[0] * r + p[1] * gg + p[2] * b + p[9])
            o1_v[buf][s] = _fold(p[3] * r + p[4] * gg + p[5] * b + p[10])
            o2_v[buf][s] = _fold(p[6] * r + p[7] * gg + p[8] * b + p[11])

    def start_in(j, buf):
        cs = pl.ds(worker_base + j * _CHUNK, _CHUNK)
        return [pltpu.async_copy(idx_hbm.at[cs], idx_v[buf], in_sems[buf]),
                pltpu.async_copy(r_hbm.at[cs], r_v[buf], in_sems[buf]),
                pltpu.async_copy(g_hbm.at[cs], g_v[buf], in_sems[buf]),
                pltpu.async_copy(b_hbm.at[cs], b_v[buf], in_sems[buf])]

    def start_out(j, buf):
        cs = pl.ds(worker_base + j * _CHUNK, _CHUNK)
        return [pltpu.async_copy(o0_v[buf], o0_hbm.at[cs], out_sems[buf]),
                pltpu.async_copy(o1_v[buf], o1_hbm.at[cs], out_sems[buf]),
                pltpu.async_copy(o2_v[buf], o2_hbm.at[cs], out_sems[buf])]

    in_flight = start_in(0, 0)
    pltpu.sync_copy(table_hbm, table_v)
    out_flight = [None, None]
    for j in range(chunks):
        buf = j % 2
        cur_in = in_flight
        if j + 1 < chunks:
            in_flight = start_in(j + 1, 1 - buf)
        for h in cur_in:
            h.wait()
        if out_flight[buf] is not None:
            for h in out_flight[buf]:
                h.wait()
        run_groups(buf)
        out_flight[buf] = start_out(j, buf)
    for hs in out_flight:
        if hs is not None:
            for h in hs:
                h.wait()


def kernel(rgb, image_indices, imageids_to_bias, imageids_to_full_matrix):
    n = rgb.shape[0]
    m = imageids_to_full_matrix.shape[0]
    table = jnp.concatenate(
        [imageids_to_full_matrix.reshape(m, 9),
         imageids_to_bias.reshape(m, 3)], axis=1)
    table = jax.lax.bitcast_convert_type(
        table.astype(jnp.bfloat16).reshape(m, 6, 2), jnp.int32).reshape(-1)
    idx = image_indices.astype(jnp.int32)

    mesh = plsc.VectorSubcoreMesh(core_axis_name="c", subcore_axis_name="s")
    plane = jax.ShapeDtypeStruct((n,), jnp.float32)
    run = pl.kernel(
        functools.partial(_sc_body, n),
        out_type=(plane, plane, plane),
        mesh=mesh,
        scratch_types=[
            pltpu.VMEM((6 * m,), jnp.int32),
            [pltpu.VMEM((_CHUNK,), jnp.int32)] * 2,
        ] + [[pltpu.VMEM((_CHUNK,), jnp.float32)] * 2] * 6
          + [[pltpu.SemaphoreType.DMA] * 2] * 2,
        compiler_params=pltpu.CompilerParams(needs_layout_passes=False),
    )
    o0, o1, o2 = run(table, rgb[:, 0], rgb[:, 1], rgb[:, 2], idx)
    return jnp.stack([o0, o1, o2], axis=1)


# speedup vs baseline: 1.0237x; 1.0237x over previous
import functools

import jax
import jax.numpy as jnp
from jax import lax
from jax.experimental import pallas as pl
from jax.experimental.pallas import tpu as pltpu
from jax.experimental.pallas import tpu_sc as plsc

_NUM_CORES = 2
_NUM_SUBCORES = 16
_LANES = 16
_NW = _NUM_CORES * _NUM_SUBCORES

_CHUNK = 8192


def _fold(x):
    u = jnp.abs(lax.rem(x, jnp.float32(2.0)))
    return jnp.minimum(u, jnp.float32(2.0) - u)


def _sc_body(n_rays, table_hbm, r_hbm, g_hbm, b_hbm, idx_hbm,
             o0_hbm, o1_hbm, o2_hbm,
             table_v, idx_v, r_v, g_v, b_v, o0_v, o1_v, o2_v,
             in_sems, out_sems):
    rays_per_worker = n_rays // _NW
    chunks = rays_per_worker // _CHUNK
    groups = _CHUNK // _LANES

    wid = lax.axis_index("s") * _NUM_CORES + lax.axis_index("c")
    worker_base = wid * rays_per_worker

    def run_groups(buf):
        @plsc.parallel_loop(0, groups, step=1, unroll=4)
        def _(g):
            s = pl.ds(g * _LANES, _LANES)
            vidx = idx_v[buf][s]
            base = vidx * 12
            p = [plsc.load_gather(table_v, [base + k]) for k in range(12)]
            r = r_v[buf][s]
            gg = g_v[buf][s]
            b = b_v[buf][s]
            o0_v[buf][s] = _fold(p[0] * r + p[1] * gg + p[2] * b + p[9])
            o1_v[buf][s] = _fold(p[3] * r + p[4] * gg + p[5] * b + p[10])
            o2_v[buf][s] = _fold(p[6] * r + p[7] * gg + p[8] * b + p[11])

    def start_in(j, buf):
        cs = pl.ds(worker_base + j * _CHUNK, _CHUNK)
        return [pltpu.async_copy(idx_hbm.at[cs], idx_v[buf], in_sems[buf]),
                pltpu.async_copy(r_hbm.at[cs], r_v[buf], in_sems[buf]),
                pltpu.async_copy(g_hbm.at[cs], g_v[buf], in_sems[buf]),
                pltpu.async_copy(b_hbm.at[cs], b_v[buf], in_sems[buf])]

    def start_out(j, buf):
        cs = pl.ds(worker_base + j * _CHUNK, _CHUNK)
        return [pltpu.async_copy(o0_v[buf], o0_hbm.at[cs], out_sems[buf]),
                pltpu.async_copy(o1_v[buf], o1_hbm.at[cs], out_sems[buf]),
                pltpu.async_copy(o2_v[buf], o2_hbm.at[cs], out_sems[buf])]

    in_flight = start_in(0, 0)
    pltpu.sync_copy(table_hbm, table_v)
    out_flight = [None, None]
    for j in range(chunks):
        buf = j % 2
        cur_in = in_flight
        if j + 1 < chunks:
            in_flight = start_in(j + 1, 1 - buf)
        for h in cur_in:
            h.wait()
        if out_flight[buf] is not None:
            for h in out_flight[buf]:
                h.wait()
        run_groups(buf)
        out_flight[buf] = start_out(j, buf)
    for hs in out_flight:
        if hs is not None:
            for h in hs:
                h.wait()


def kernel(rgb, image_indices, imageids_to_bias, imageids_to_full_matrix):
    n = rgb.shape[0]
    m = imageids_to_full_matrix.shape[0]
    table = jnp.concatenate(
        [imageids_to_full_matrix.reshape(m, 9),
         imageids_to_bias.reshape(m, 3)], axis=1).reshape(-1)
    idx = image_indices.astype(jnp.int32)

    mesh = plsc.VectorSubcoreMesh(core_axis_name="c", subcore_axis_name="s")
    plane = jax.ShapeDtypeStruct((n,), jnp.float32)
    run = pl.kernel(
        functools.partial(_sc_body, n),
        out_type=(plane, plane, plane),
        mesh=mesh,
        scratch_types=[
            pltpu.VMEM((12 * m,), jnp.float32),
            [pltpu.VMEM((_CHUNK,), jnp.int32)] * 2,
        ] + [[pltpu.VMEM((_CHUNK,), jnp.float32)] * 2] * 6
          + [[pltpu.SemaphoreType.DMA] * 2] * 2,
        compiler_params=pltpu.CompilerParams(needs_layout_passes=False),
    )
    o0, o1, o2 = run(table, rgb[:, 0], rgb[:, 1], rgb[:, 2], idx)
    return jnp.stack([o0, o1, o2], axis=1)


# speedup vs baseline: 1.0548x; 1.0304x over previous
import functools

import jax
import jax.numpy as jnp
from jax import lax
from jax.experimental import pallas as pl
from jax.experimental.pallas import tpu as pltpu
from jax.experimental.pallas import tpu_sc as plsc

_NUM_CORES = 2
_NUM_SUBCORES = 16
_LANES = 16
_NW = _NUM_CORES * _NUM_SUBCORES

_CHUNK = 8192


def _fold(x):
    u = jnp.abs(lax.rem(x, jnp.float32(2.0)))
    return jnp.where(u <= jnp.float32(1.0), u, jnp.float32(2.0) - u)


def _sc_body(n_rays, table_hbm, r_hbm, g_hbm, b_hbm, idx_hbm,
             o0_hbm, o1_hbm, o2_hbm,
             table_v, idx_v, r_v, g_v, b_v, o0_v, o1_v, o2_v,
             in_sems, out_sems):
    rays_per_worker = n_rays // _NW
    chunks = rays_per_worker // _CHUNK
    groups = _CHUNK // _LANES

    wid = lax.axis_index("s") * _NUM_CORES + lax.axis_index("c")
    worker_base = wid * rays_per_worker

    def run_groups(buf):
        @plsc.parallel_loop(0, groups, step=1, unroll=4)
        def _(g):
            s = pl.ds(g * _LANES, _LANES)
            vidx = idx_v[buf][s]
            base = vidx * 12
            p = [plsc.load_gather(table_v, [base + k]) for k in range(12)]
            r = r_v[buf][s]
            gg = g_v[buf][s]
            b = b_v[buf][s]
            o0_v[buf][s] = _fold(p[0] * r + p[1] * gg + p[2] * b + p[9])
            o1_v[buf][s] = _fold(p[3] * r + p[4] * gg + p[5] * b + p[10])
            o2_v[buf][s] = _fold(p[6] * r + p[7] * gg + p[8] * b + p[11])

    def start_in(j, buf):
        cs = pl.ds(worker_base + j * _CHUNK, _CHUNK)
        return [pltpu.async_copy(idx_hbm.at[cs], idx_v[buf], in_sems[buf]),
                pltpu.async_copy(r_hbm.at[cs], r_v[buf], in_sems[buf]),
                pltpu.async_copy(g_hbm.at[cs], g_v[buf], in_sems[buf]),
                pltpu.async_copy(b_hbm.at[cs], b_v[buf], in_sems[buf])]

    def start_out(j, buf):
        cs = pl.ds(worker_base + j * _CHUNK, _CHUNK)
        return [pltpu.async_copy(o0_v[buf], o0_hbm.at[cs], out_sems[buf]),
                pltpu.async_copy(o1_v[buf], o1_hbm.at[cs], out_sems[buf]),
                pltpu.async_copy(o2_v[buf], o2_hbm.at[cs], out_sems[buf])]

    in_flight = start_in(0, 0)
    pltpu.sync_copy(table_hbm, table_v)
    out_flight = [None, None]
    for j in range(chunks):
        buf = j % 2
        cur_in = in_flight
        if j + 1 < chunks:
            in_flight = start_in(j + 1, 1 - buf)
        for h in cur_in:
            h.wait()
        if out_flight[buf] is not None:
            for h in out_flight[buf]:
                h.wait()
        run_groups(buf)
        out_flight[buf] = start_out(j, buf)
    for hs in out_flight:
        if hs is not None:
            for h in hs:
                h.wait()


def kernel(rgb, image_indices, imageids_to_bias, imageids_to_full_matrix):
    n = rgb.shape[0]
    m = imageids_to_full_matrix.shape[0]
    table = jnp.concatenate(
        [imageids_to_full_matrix.reshape(m, 9),
         imageids_to_bias.reshape(m, 3)], axis=1).reshape(-1)
    idx = image_indices.astype(jnp.int32)

    mesh = plsc.VectorSubcoreMesh(core_axis_name="c", subcore_axis_name="s")
    plane = jax.ShapeDtypeStruct((n,), jnp.float32)
    run = pl.kernel(
        functools.partial(_sc_body, n),
        out_type=(plane, plane, plane),
        mesh=mesh,
        scratch_types=[
            pltpu.VMEM((12 * m,), jnp.float32),
            [pltpu.VMEM((_CHUNK,), jnp.int32)] * 2,
        ] + [[pltpu.VMEM((_CHUNK,), jnp.float32)] * 2] * 6
          + [[pltpu.SemaphoreType.DMA] * 2] * 2,
        compiler_params=pltpu.CompilerParams(needs_layout_passes=False),
    )
    o0, o1, o2 = run(table, rgb[:, 0], rgb[:, 1], rgb[:, 2], idx)
    return jnp.stack([o0, o1, o2], axis=1)
